# baseline (speedup 1.0000x reference)
import functools

import jax
import jax.numpy as jnp
from jax import lax
from jax.experimental import pallas as pl
from jax.experimental.pallas import tpu as pltpu
from jax.experimental.pallas import tpu_sc as plsc

DIM = 64
NW = 32
CHUNK = 1024
SEG = 128
STREAMS = CHUNK // SEG


def _make_gather(n_idx):
    per_w = n_idx // NW
    n_chunk = per_w // CHUNK
    mesh = plsc.VectorSubcoreMesh(core_axis_name="c", subcore_axis_name="s")

    @functools.partial(
        pl.kernel,
        mesh=mesh,
        out_type=jax.ShapeDtypeStruct((n_idx, DIM), jnp.float32),
        scratch_types=[
            pltpu.VMEM((STREAMS, SEG), jnp.int32),
            pltpu.VMEM((CHUNK, DIM), jnp.float32),
            pltpu.SemaphoreType.DMA,
        ],
        compiler_params=pltpu.CompilerParams(use_tc_tiling_on_sc=False),
    )
    def gather(idx_hbm, table_hbm, out_hbm, idx_v, rows_v, sem):
        wid = lax.axis_index("s") * 2 + lax.axis_index("c")
        row_base = wid * (per_w // SEG)

        def body(i, _):
            off = (row_base + i * STREAMS) * SEG
            pltpu.sync_copy(
                idx_hbm.at[pl.ds(row_base + i * STREAMS, STREAMS)], idx_v)
            copies = [
                pltpu.async_copy(
                    table_hbm.at[idx_v.at[j]],
                    rows_v.at[pl.ds(j * SEG, SEG)],
                    sem,
                )
                for j in range(STREAMS)
            ]
            for c in copies:
                c.wait()
            pltpu.sync_copy(rows_v, out_hbm.at[pl.ds(off, CHUNK)])
            return _

        lax.fori_loop(0, n_chunk, body, None)

    return gather


def kernel(token_id, table):
    b, l = token_id.shape
    n = b * l
    idx2d = token_id.reshape(n // SEG, SEG).astype(jnp.int32)
    out = _make_gather(n)(idx2d, table)
    return out.reshape(b, l, DIM)


# speedup vs baseline: 1.0169x; 1.0169x over previous
import functools

import jax
import jax.numpy as jnp
from jax import lax
from jax.experimental import pallas as pl
from jax.experimental.pallas import tpu as pltpu
from jax.experimental.pallas import tpu_sc as plsc

DIM = 64
NW = 32
SEG = 128
CHUNK = 512
STREAMS = CHUNK // SEG


def _make_gather(n_idx):
    per_w = n_idx // NW
    seg_per_w = per_w // SEG
    n_chunk = per_w // CHUNK
    assert n_chunk % 2 == 0 and n_chunk >= 4
    mesh = plsc.VectorSubcoreMesh(core_axis_name="c", subcore_axis_name="s")

    @functools.partial(
        pl.kernel,
        mesh=mesh,
        out_type=jax.ShapeDtypeStruct((n_idx, DIM), jnp.float32),
        scratch_types=[
            pltpu.VMEM((seg_per_w, SEG), jnp.int32),
            pltpu.VMEM((2, CHUNK, DIM), jnp.float32),
            pltpu.SemaphoreType.DMA,
            pltpu.SemaphoreType.DMA,
            pltpu.SemaphoreType.DMA,
            pltpu.SemaphoreType.DMA,
        ],
        compiler_params=pltpu.CompilerParams(use_tc_tiling_on_sc=False),
    )
    def gather(idx_hbm, table_hbm, out_hbm, idx_v, rows_v, g0, g1, s0, s1):
        gsem = (g0, g1)
        ssem = (s0, s1)
        wid = lax.axis_index("s") * 2 + lax.axis_index("c")
        base = wid * per_w

        pltpu.sync_copy(idx_hbm.at[pl.ds(wid * seg_per_w, seg_per_w)], idx_v)

        def fire_gathers(g, b):
            for j in range(STREAMS):
                pltpu.async_copy(
                    table_hbm.at[idx_v.at[g * STREAMS + j]],
                    rows_v.at[b].at[pl.ds(j * SEG, SEG)],
                    gsem[b],
                )

        def wait_gathers(b):
            pltpu.make_async_copy(
                table_hbm.at[pl.ds(0, CHUNK)], rows_v.at[b], gsem[b]
            ).wait()

        def fire_store(g, b):
            pltpu.async_copy(
                rows_v.at[b], out_hbm.at[pl.ds(base + g * CHUNK, CHUNK)],
                ssem[b],
            )

        def wait_store(b):
            pltpu.make_async_copy(
                rows_v.at[b], out_hbm.at[pl.ds(base, CHUNK)], ssem[b]
            ).wait()

        fire_gathers(0, 0)
        fire_gathers(1, 1)

        def body(p, _):
            g = 2 * p + 2
            for b in (0, 1):
                wait_gathers(1 - b)
                fire_store(g - 1, 1 - b)
                wait_store(b)
                fire_gathers(g, b)
                g = g + 1
            return _

        wait_gathers(0)
        fire_store(0, 0)
        lax.fori_loop(0, (n_chunk - 2) // 2, body, None)

        wait_gathers(1)
        fire_store(n_chunk - 1, 1)
        wait_store(0)
        wait_store(1)

    return gather


def kernel(token_id, table):
    b, l = token_id.shape
    n = b * l
    idx2d = token_id.reshape(n // SEG, SEG).astype(jnp.int32)
    out = _make_gather(n)(idx2d, table)
    return out.reshape(b, l, DIM)
